# baseline (speedup 1.0000x reference)
import functools

import jax
import jax.numpy as jnp
from jax import lax
from jax.experimental import pallas as pl
from jax.experimental.pallas import tpu as pltpu
from jax.experimental.pallas import tpu_sc as plsc

T = 160000
R = 10000
NB = 4
D = 128
NW = 32
UNROLL = 8
CHUNK = T // NW
CPAD = CHUNK + 8
NV = CHUNK // (16 * UNROLL)
TAIL = CHUNK - NV * 16 * UNROLL
QCAP = 2 * CHUNK + 16
CG = (2 * R * NB) // D

_mesh = plsc.VectorSubcoreMesh(core_axis_name="c", subcore_axis_name="s")


@functools.partial(
    pl.kernel,
    mesh=_mesh,
    compiler_params=pltpu.CompilerParams(needs_layout_passes=False),
    out_type=[
        jax.ShapeDtypeStruct((NW, NB, 2 * D), jnp.float32),
        jax.ShapeDtypeStruct((NW, 16), jnp.float32),
    ],
    scratch_types=[
        pltpu.VMEM((CPAD,), jnp.int32),
        pltpu.VMEM((CPAD,), jnp.int32),
        pltpu.VMEM((CPAD,), jnp.int32),
        pltpu.VMEM((16,), jnp.int32),
        pltpu.VMEM((QCAP,), jnp.int32),
        pltpu.VMEM((QCAP,), jnp.int32),
        pltpu.VMEM((QCAP,), jnp.int32),
        pltpu.VMEM((4 * 16,), jnp.int32),
        pltpu.VMEM((NB, 2 * D), jnp.float32),
        pltpu.VMEM((16, D), jnp.float32),
        pltpu.VMEM((16, D), jnp.float32),
        pltpu.VMEM((4 * 16,), jnp.float32),
        pltpu.VMEM((16,), jnp.float32),
        pltpu.SemaphoreType.DMA,
    ],
)
def _sc_scan(tri_hbm, u_hbm, ent_hbm, rel_hbm, cflat_hbm, part_out, cnt_out,
             s_v, r_v, d_v, u_v, qe_v, qr_v, qc_v, qcat_v, s_acc, ebuf, rbuf,
             ccat_v, cntf_v, sem):
    wid = lax.axis_index("s") * 2 + lax.axis_index("c")
    base = wid * CHUNK
    cps = pltpu.async_copy(tri_hbm.at[pl.ds(base, CHUNK)],
                           s_v.at[pl.ds(0, CHUNK)], sem)
    cpr = pltpu.async_copy(tri_hbm.at[pl.ds(T + base, CHUNK)],
                           r_v.at[pl.ds(0, CHUNK)], sem)
    cpd = pltpu.async_copy(tri_hbm.at[pl.ds(2 * T + base, CHUNK)],
                           d_v.at[pl.ds(0, CHUNK)], sem)
    pltpu.sync_copy(u_hbm, u_v)
    cps.wait()
    cpr.wait()
    cpd.wait()

    zeros16 = jnp.zeros((16,), jnp.float32)
    for b in range(NB):
        for k in range(2 * D // 16):
            s_acc[b, pl.ds(k * 16, 16)] = zeros16

    lane16 = lax.iota(jnp.int32, 16)
    rsplat = jnp.full((16,), R, jnp.int32)

    uvh = u_v[...]

    def scan_body(i, cnt):
        off = i * (16 * UNROLL)
        uv = uvh
        svs = [s_v[pl.ds(off + 16 * h, 16)] for h in range(UNROLL)]
        dvs = [d_v[pl.ds(off + 16 * h, 16)] for h in range(UNROLL)]
        mfs = [dvs[h] == uv for h in range(UNROLL)]
        mrs = [svs[h] == uv for h in range(UNROLL)]
        many = mfs[0]
        for h in range(UNROLL):
            many = jnp.logical_or(many, mfs[h]) if h else many
            many = jnp.logical_or(many, mrs[h])
        nt = plsc.all_reduce_population_count(many)[0]

        @pl.when(nt > 0)
        def _():
            def h_body(h, c):
                hoff = off + 16 * h
                svh = s_v[pl.ds(hoff, 16)]
                rvh = r_v[pl.ds(hoff, 16)]
                dvh = d_v[pl.ds(hoff, 16)]
                mfh = dvh == uvh
                mrh = svh == uvh
                nfh = plsc.all_reduce_population_count(mfh)[0]
                nrh = plsc.all_reduce_population_count(mrh)[0]
                plsc.store_compressed(qe_v.at[pl.ds(c, 16)], svh, mask=mfh)
                plsc.store_compressed(qr_v.at[pl.ds(c, 16)], rvh, mask=mfh)
                plsc.store_compressed(qc_v.at[pl.ds(c, 16)], rvh, mask=mfh)
                c2 = c + nfh
                plsc.store_compressed(qe_v.at[pl.ds(c2, 16)], dvh, mask=mrh)
                plsc.store_compressed(qr_v.at[pl.ds(c2, 16)], rvh, mask=mrh)
                plsc.store_compressed(qc_v.at[pl.ds(c2, 16)], rvh + rsplat,
                                      mask=mrh)
                return c2 + nrh

            lax.fori_loop(0, UNROLL, h_body, cnt)

        return cnt + nt

    n = lax.fori_loop(0, NV, scan_body, jnp.int32(0))

    toff0 = NV * 16 * UNROLL
    uv0 = uvh
    svt = s_v[pl.ds(toff0, 16)]
    rvt = r_v[pl.ds(toff0, 16)]
    dvt = d_v[pl.ds(toff0, 16)]
    lvalid = lane16 < jnp.full((16,), TAIL, jnp.int32)
    mft = jnp.logical_and(dvt == uv0, lvalid)
    mrt = jnp.logical_and(svt == uv0, lvalid)
    ntt = plsc.all_reduce_population_count(jnp.logical_or(mft, mrt))[0]
    npre = n

    @pl.when(ntt > 0)
    def _():
        nft = plsc.all_reduce_population_count(mft)[0]
        plsc.store_compressed(qe_v.at[pl.ds(npre, 16)], svt, mask=mft)
        plsc.store_compressed(qr_v.at[pl.ds(npre, 16)], rvt, mask=mft)
        plsc.store_compressed(qc_v.at[pl.ds(npre, 16)], rvt, mask=mft)
        c2t = npre + nft
        plsc.store_compressed(qe_v.at[pl.ds(c2t, 16)], dvt, mask=mrt)
        plsc.store_compressed(qr_v.at[pl.ds(c2t, 16)], rvt, mask=mrt)
        plsc.store_compressed(qc_v.at[pl.ds(c2t, 16)], rvt + rsplat,
                              mask=mrt)

    n = n + ntt

    nb = (n + 15) // 16

    @pl.when(n > 0)
    def _():
        toff = (nb - 1) * 16
        valid_tail = (lane16 + jnp.full((16,), toff, jnp.int32)) < jnp.full(
            (16,), n, jnp.int32)
        for q in (qe_v, qr_v, qc_v):
            qv = q[pl.ds(toff, 16)]
            q[pl.ds(toff, 16)] = jnp.where(valid_tail, qv,
                                           jnp.zeros((16,), jnp.int32))

    def batch_body(j, _):
        qoff = j * 16
        qrow = qc_v[pl.ds(qoff, 16)]
        for b in range(NB):
            qcat_v[pl.ds(16 * b, 16)] = qrow + jnp.full((16,), b * 2 * R,
                                                        jnp.int32)
        cp_e = pltpu.async_copy(ent_hbm.at[qe_v.at[pl.ds(qoff, 16)]], ebuf,
                                sem)
        cp_r = pltpu.async_copy(rel_hbm.at[qr_v.at[pl.ds(qoff, 16)]], rbuf,
                                sem)
        cp_c = pltpu.async_copy(cflat_hbm.at[qcat_v], ccat_v, sem)
        cp_e.wait()
        cp_r.wait()
        cp_c.wait()

        def edge_step(e, _unused):
            esplat = jnp.full((16,), e, jnp.int32)
            cbs = [plsc.load_gather(ccat_v, [esplat + jnp.full(
                (16,), 16 * b, jnp.int32)]) for b in range(NB)]
            for k in range(D // 16):
                ev = ebuf[e, pl.ds(k * 16, 16)]
                rv2 = rbuf[e, pl.ds(k * 16, 16)]
                for b in range(NB):
                    s_acc[b, pl.ds(k * 16, 16)] += cbs[b] * ev
                    s_acc[b, pl.ds(D + k * 16, 16)] += cbs[b] * rv2
            return 0

        ec = jnp.minimum(n - qoff, 16)
        lax.fori_loop(0, ec, edge_step, 0)
        return 0

    lax.fori_loop(0, nb, batch_body, 0)

    pltpu.sync_copy(s_acc, part_out.at[wid])
    cntf_v[...] = jnp.full((16,), n.astype(jnp.float32), jnp.float32)
    pltpu.sync_copy(cntf_v, cnt_out.at[wid])


def _tc_body(part_ref, cnt_ref, bases_ref, out_ref):
    s_total = jnp.sum(part_ref[...], axis=0)
    total = jnp.sum(cnt_ref[...]) * (1.0 / 16.0)
    denom = jnp.maximum(total, 1.0)
    acc = jnp.zeros((1, D), jnp.float32)
    for b in range(NB):
        acc = acc + jnp.dot(s_total[b:b + 1, :], bases_ref[b],
                            preferred_element_type=jnp.float32)
    out_ref[...] = acc / denom


def kernel(unseen_entity, triplets, use_cuda, entity_table, relation_table,
           bases, comp):
    trip = jnp.asarray(triplets).astype(jnp.int32)
    u_arr = jnp.full((16,), jnp.asarray(unseen_entity, jnp.int32))
    c_flat = comp.astype(jnp.float32).T.reshape(NB * 2 * R)

    tri_flat = trip.T.reshape(3 * T)
    part, cnt = _sc_scan(tri_flat, u_arr, entity_table, relation_table,
                         c_flat)

    out = pl.pallas_call(
        _tc_body,
        out_shape=jax.ShapeDtypeStruct((1, D), jnp.float32),
    )(part, cnt, bases)
    return out.reshape(D)
